# Initial kernel scaffold; baseline (speedup 1.0000x reference)
import jax
import jax.numpy as jnp
from jax.experimental import pallas as pl


def kernel(pred, target, feat, batch, centers):
    raise NotImplementedError("write your pallas kernel here")


# speedup vs baseline: 5.9187x; 5.9187x over previous
import functools

import jax
import jax.numpy as jnp
from jax import lax
from jax.experimental import pallas as pl
from jax.experimental.pallas import tpu as pltpu
from jax.experimental.pallas import tpu_sc as plsc

N = 262144
D = 64
C = 16
B = 8
MARGIN_ = 0.5
LOSS_W_ = 0.01

NCORES = 2
NSUB = 16
NW = NCORES * NSUB
PW = N // NW
CH = 1024
NCH = PW // CH
ROWS128 = CH // 128


def _sc_stage1(feat, tgt2, bat2):
    mesh = plsc.VectorSubcoreMesh(core_axis_name="c", subcore_axis_name="s")
    out_type = jax.ShapeDtypeStruct((NCORES, B * C, D), jnp.float32)
    scratch = [
        pltpu.VMEM((CH // 2, D), jnp.float32),
        pltpu.VMEM((ROWS128, 128), jnp.int32),
        pltpu.VMEM((ROWS128, 128), jnp.int32),
        pltpu.VMEM((ROWS128, 128), jnp.int32),
        pltpu.VMEM((8, D), jnp.float32),
        pltpu.VMEM_SHARED((B * C, D), jnp.float32),
    ]

    @functools.partial(pl.kernel, out_type=out_type, mesh=mesh,
                       scratch_types=scratch)
    def k(feat_h, tgt_h, bat_h, fs_o, featv, tgtv, batv, segv, zv, fs_sh):
        cid = lax.axis_index("c")
        sid = lax.axis_index("s")
        zero16 = jnp.zeros((16,), jnp.float32)

        for r in range(8):
            for j in range(D // 16):
                zv[r, pl.ds(16 * j, 16)] = zero16

        pltpu.sync_copy(zv, fs_sh.at[pl.ds(sid * 8, 8)])
        plsc.subcore_barrier()

        wid = cid * NSUB + sid
        base = wid * PW

        def chunk(kk, carry):
            pt0 = pl.multiple_of(base + kk * CH, CH)
            rr = base // CH + kk
            pltpu.sync_copy(tgt_h.at[rr], tgtv)
            pltpu.sync_copy(bat_h.at[rr], batv)
            for r in range(ROWS128):
                for j in range(8):
                    sl = pl.ds(16 * j, 16)
                    segv[r, sl] = batv[r, sl] * C + tgtv[r, sl]
            for h in range(2):
                pltpu.sync_copy(feat_h.at[pl.ds(pt0 + h * (CH // 2), CH // 2)],
                                featv)
                for g in range(ROWS128 // 2):
                    pltpu.sync_copy(featv.at[pl.ds(g * 128, 128)],
                                    fs_sh.at[segv.at[h * (ROWS128 // 2) + g]],
                                    add=True)
            return carry
        lax.fori_loop(0, NCH, chunk, 0)

        plsc.subcore_barrier()

        @pl.when(sid == 0)
        def _():
            pltpu.sync_copy(fs_sh, fs_o.at[cid])

    return k(feat, tgt2, bat2)


_BLK = 16384
_NBLK = N // _BLK


def _sumsq_body(feat_ref, bat_ref, tgt_ref, sq_ref, cnt_ref):
    f = feat_ref[...]
    rs = jnp.sum(f * f, axis=1, keepdims=True)
    bids = bat_ref[...]
    tids = tgt_ref[...]
    ohb = (bids == lax.broadcasted_iota(jnp.int32, (1, B), 1)).astype(jnp.float32)
    sq_ref[...] = jnp.sum(ohb * rs, axis=0).reshape(1, 1, B)
    seg = bids * C + tids
    ohs = (seg == lax.broadcasted_iota(jnp.int32, (1, B * C), 1)).astype(jnp.float32)
    cnt_col = lax.dot_general(ohs, jnp.ones((_BLK, 1), jnp.float32),
                              (((0,), (0,)), ((), ())))
    cnt_ref[...] = cnt_col.reshape(1, B * C, 1)


def _tc_sumsq(feat, bat_col, tgt_col):
    return pl.pallas_call(
        _sumsq_body,
        grid=(_NBLK,),
        in_specs=[
            pl.BlockSpec((_BLK, D), lambda i: (i, 0)),
            pl.BlockSpec((_BLK, 1), lambda i: (i, 0)),
            pl.BlockSpec((_BLK, 1), lambda i: (i, 0)),
        ],
        out_specs=[
            pl.BlockSpec((1, 1, B), lambda i: (i, 0, 0)),
            pl.BlockSpec((1, B * C, 1), lambda i: (i, 0, 0)),
        ],
        out_shape=[
            jax.ShapeDtypeStruct((_NBLK, 1, B), jnp.float32),
            jax.ShapeDtypeStruct((_NBLK, B * C, 1), jnp.float32),
        ],
    )(feat, bat_col, tgt_col)


def _dot00(a, b):
    return lax.dot_general(a, b, (((0,), (0,)), ((), ())))


def _dot11(a, b):
    return lax.dot_general(a, b, (((1,), (1,)), ((), ())))


def _final_body(fs_ref, cnt_ref, sqp_ref, cen_ref, out_ref):
    fs2 = fs_ref[...]
    fs = fs2[0] + fs2[1]
    cnt = jnp.sum(cnt_ref[...], axis=0)
    sqb = jnp.sum(sqp_ref[...], axis=(0, 1)).reshape(1, B)
    cen = cen_ref[...]
    cen_t = jnp.concatenate([cen] * B, axis=0)

    rowdot = jnp.sum(fs * cen_t, axis=1, keepdims=True)
    rowcn2 = jnp.sum(cen_t * cen_t, axis=1, keepdims=True)
    ohg = (lax.broadcasted_iota(jnp.int32, (B * C, B), 0) // C
           == lax.broadcasted_iota(jnp.int32, (B * C, B), 1)).astype(jnp.float32)
    dotterm = _dot00(rowdot, ohg)
    cterm = _dot00(cnt * rowcn2, ohg)
    cnt_b = _dot00(cnt, ohg)
    intra_sum = sqb - 2.0 * dotterm + cterm
    has_b = cnt_b > 0
    intra_b = jnp.where(has_b, intra_sum / jnp.maximum(cnt_b, 1.0), 0.0)
    total_intra = jnp.sum(intra_b)
    batch_count = jnp.sum(has_b.astype(jnp.float32))

    ri = lax.broadcasted_iota(jnp.int32, (C, C), 0)
    ci = lax.broadcasted_iota(jnp.int32, (C, C), 1)
    eye_f = (ri == ci).astype(jnp.float32)
    noteye_f = 1.0 - eye_f
    total_inter = jnp.float32(0.0)
    for b in range(B):
        ccnt = cnt[b * C:(b + 1) * C, :]
        cc = fs[b * C:(b + 1) * C, :] / jnp.maximum(ccnt, 1.0)
        g = _dot11(cc, cc)
        ncol = jnp.sum(g * eye_f, axis=1, keepdims=True)
        nrow = jnp.sum(g * eye_f, axis=0, keepdims=True)
        sq = ncol + nrow - 2.0 * g
        pos = sq > 0
        dist = jnp.where(pos, jnp.sqrt(jnp.where(pos, sq, 1.0)), 0.0)
        pres = (ccnt > 0).astype(jnp.float32)
        pm = _dot11(pres, pres) * noteye_f
        hinge = jnp.maximum(MARGIN_ - dist, 0.0)
        n_pairs = jnp.sum(pm)
        n_present = jnp.sum(pres)
        inter_b = jnp.where(
            n_present > 1,
            jnp.sum(hinge * pm) / jnp.maximum(n_pairs, 1.0),
            0.0)
        total_inter = total_inter + inter_b

    avg_intra = jnp.where(batch_count > 0,
                          total_intra / jnp.maximum(batch_count, 1.0), 0.0)
    avg_inter = jnp.where(batch_count > 0,
                          total_inter / jnp.maximum(batch_count, 1.0), 0.0)
    out_ref[...] = (LOSS_W_ * (avg_intra + avg_inter)).reshape(1, 1)


def _tc_finalize(fs, cnt, sqp, centers):
    return pl.pallas_call(
        _final_body,
        out_shape=jax.ShapeDtypeStruct((1, 1), jnp.float32),
    )(fs, cnt, sqp, centers)


def kernel(pred, target, feat, batch, centers):
    del pred
    tgt2 = target.reshape(N // CH, ROWS128, 128)
    bat2 = batch.reshape(N // CH, ROWS128, 128)
    fs = _sc_stage1(feat, tgt2, bat2)
    sqp, cntp = _tc_sumsq(feat, batch.reshape(N, 1), target.reshape(N, 1))
    out = _tc_finalize(fs, cntp, sqp, centers)
    return out[0, 0]


# speedup vs baseline: 5.9780x; 1.0100x over previous
import functools

import jax
import jax.numpy as jnp
from jax import lax
from jax.experimental import pallas as pl
from jax.experimental.pallas import tpu as pltpu
from jax.experimental.pallas import tpu_sc as plsc

N = 262144
D = 64
C = 16
B = 8
MARGIN_ = 0.5
LOSS_W_ = 0.01

NCORES = 2
NSUB = 16
NW = NCORES * NSUB
PW = N // NW
IDXROWS = PW // 128
CHP = 256
NSC = PW // CHP
GPC = CHP // 128


def _sc_stage1(feat, tgt2, bat2):
    mesh = plsc.VectorSubcoreMesh(core_axis_name="c", subcore_axis_name="s")
    out_type = jax.ShapeDtypeStruct((NCORES, B * C, D), jnp.float32)
    scratch = [
        pltpu.VMEM((CHP, D), jnp.float32),
        pltpu.VMEM((CHP, D), jnp.float32),
        pltpu.VMEM((IDXROWS, 128), jnp.int32),
        pltpu.VMEM((IDXROWS, 128), jnp.int32),
        pltpu.VMEM((IDXROWS, 128), jnp.int32),
        pltpu.VMEM((8, D), jnp.float32),
        pltpu.VMEM_SHARED((B * C, D), jnp.float32),
        pltpu.SemaphoreType.DMA,
        pltpu.SemaphoreType.DMA,
        pltpu.SemaphoreType.DMA,
        pltpu.SemaphoreType.DMA,
    ]

    @functools.partial(pl.kernel, out_type=out_type, mesh=mesh,
                       scratch_types=scratch)
    def k(feat_h, tgt_h, bat_h, fs_o, fA, fB, tgtv, batv, segv, zv, fs_sh,
          lsemA, lsemB, ssemA, ssemB):
        cid = lax.axis_index("c")
        sid = lax.axis_index("s")
        zero16 = jnp.zeros((16,), jnp.float32)

        for r in range(8):
            for j in range(D // 16):
                zv[r, pl.ds(16 * j, 16)] = zero16

        pltpu.sync_copy(zv, fs_sh.at[pl.ds(sid * 8, 8)])

        wid = cid * NSUB + sid
        base = wid * PW

        row0 = pl.multiple_of(wid * IDXROWS, 8)
        pltpu.sync_copy(tgt_h.at[pl.ds(row0, IDXROWS)], tgtv)
        pltpu.sync_copy(bat_h.at[pl.ds(row0, IDXROWS)], batv)

        def segrow(r, carry):
            for j in range(8):
                sl = pl.ds(16 * j, 16)
                segv[r, sl] = batv[r, sl] * C + tgtv[r, sl]
            return carry
        lax.fori_loop(0, IDXROWS, segrow, 0)
        plsc.subcore_barrier()

        def feat_src(c):
            pt0 = pl.multiple_of(base + c * CHP, CHP)
            return feat_h.at[pl.ds(pt0, CHP)]

        def ld_start(c, buf, sem):
            pltpu.async_copy(feat_src(c), buf, sem)

        def ld_wait(c, buf, sem):
            pltpu.make_async_copy(feat_src(c), buf, sem).wait()

        def scat_start(c, buf, sem):
            for g in range(GPC):
                pltpu.async_copy(buf.at[pl.ds(g * 128, 128)],
                                 fs_sh.at[segv.at[c * GPC + g]], sem, add=True)

        def scat_wait(c, buf, sem):
            for g in range(GPC):
                pltpu.make_async_copy(buf.at[pl.ds(g * 128, 128)],
                                      fs_sh.at[segv.at[c * GPC + g]],
                                      sem).wait()

        ld_start(0, fA, lsemA)

        def pipe(i, carry):
            for b in range(2):
                c = i * 2 + b
                buf, lsem, ssem = (fA, lsemA, ssemA) if b == 0 else (fB, lsemB, ssemB)
                obuf, olsem, ossem = (fB, lsemB, ssemB) if b == 0 else (fA, lsemA, ssemA)
                ld_wait(c, buf, lsem)

                @pl.when(c >= 1)
                def _():
                    scat_wait(c - 1, obuf, ossem)

                @pl.when(c + 1 < NSC)
                def _():
                    ld_start(c + 1, obuf, olsem)

                scat_start(c, buf, ssem)
            return carry
        lax.fori_loop(0, NSC // 2, pipe, 0)
        scat_wait(NSC - 1, fB, ssemB)

        plsc.subcore_barrier()

        @pl.when(sid == 0)
        def _():
            pltpu.sync_copy(fs_sh, fs_o.at[cid])

    return k(feat, tgt2, bat2)


_BLK = 8192
_NBLK = N // _BLK


def _sumsq_body(feat_ref, bat_ref, tgt_ref, sq_ref, cnt_ref):
    f = feat_ref[...]
    g2 = f * f
    rs = lax.dot_general(g2, jnp.ones((D, 1), jnp.float32),
                         (((1,), (0,)), ((), ())))
    bids = bat_ref[...]
    tids = tgt_ref[...]
    ohb = (bids == lax.broadcasted_iota(jnp.int32, (1, B), 1)).astype(jnp.float32)
    ohc = (tids == lax.broadcasted_iota(jnp.int32, (1, C), 1)).astype(jnp.float32)
    sq8 = lax.dot_general(ohb, rs, (((0,), (0,)), ((), ())))
    cntbc = lax.dot_general(ohb, ohc, (((0,), (0,)), ((), ())))
    sq_ref[...] = sq8.reshape(1, B, 1)
    cnt_ref[...] = cntbc.reshape(1, B, C)


def _tc_sumsq(feat, bat_col, tgt_col):
    return pl.pallas_call(
        _sumsq_body,
        grid=(_NBLK,),
        in_specs=[
            pl.BlockSpec((_BLK, D), lambda i: (i, 0)),
            pl.BlockSpec((_BLK, 1), lambda i: (i, 0)),
            pl.BlockSpec((_BLK, 1), lambda i: (i, 0)),
        ],
        out_specs=[
            pl.BlockSpec((1, B, 1), lambda i: (i, 0, 0)),
            pl.BlockSpec((1, B, C), lambda i: (i, 0, 0)),
        ],
        out_shape=[
            jax.ShapeDtypeStruct((_NBLK, B, 1), jnp.float32),
            jax.ShapeDtypeStruct((_NBLK, B, C), jnp.float32),
        ],
    )(feat, bat_col, tgt_col)


def _dot00(a, b):
    return lax.dot_general(a, b, (((0,), (0,)), ((), ())))


def _dot11(a, b):
    return lax.dot_general(a, b, (((1,), (1,)), ((), ())))


def _final_body(fs_ref, cnt_ref, sqp_ref, cen_ref, out_ref):
    fs2 = fs_ref[...]
    fs = fs2[0] + fs2[1]
    cnt_bc = jnp.sum(cnt_ref[...], axis=0)
    sqb8 = jnp.sum(sqp_ref[...], axis=0)
    cen = cen_ref[...]
    cen_t = jnp.concatenate([cen] * B, axis=0)

    ohg = (lax.broadcasted_iota(jnp.int32, (B * C, B), 0) // C
           == lax.broadcasted_iota(jnp.int32, (B * C, B), 1)).astype(jnp.float32)
    m128 = (lax.broadcasted_iota(jnp.int32, (B * C, C), 0) % C
            == lax.broadcasted_iota(jnp.int32, (B * C, C), 1)).astype(jnp.float32)
    cnt = jnp.sum(jnp.dot(ohg, cnt_bc) * m128, axis=1, keepdims=True)

    rowdot = jnp.sum(fs * cen_t, axis=1, keepdims=True)
    rowcn2 = jnp.sum(cen_t * cen_t, axis=1, keepdims=True)
    dotterm = _dot00(ohg, rowdot)
    cterm = _dot00(ohg, cnt * rowcn2)
    cnt_b = _dot00(ohg, cnt)
    intra_sum = sqb8 - 2.0 * dotterm + cterm
    has_b = cnt_b > 0
    intra_b = jnp.where(has_b, intra_sum / jnp.maximum(cnt_b, 1.0), 0.0)
    total_intra = jnp.sum(intra_b)
    batch_count = jnp.sum(has_b.astype(jnp.float32))

    ri = lax.broadcasted_iota(jnp.int32, (C, C), 0)
    ci = lax.broadcasted_iota(jnp.int32, (C, C), 1)
    eye_f = (ri == ci).astype(jnp.float32)
    noteye_f = 1.0 - eye_f
    total_inter = jnp.float32(0.0)
    for b in range(B):
        ccnt = cnt[b * C:(b + 1) * C, :]
        cc = fs[b * C:(b + 1) * C, :] / jnp.maximum(ccnt, 1.0)
        g = _dot11(cc, cc)
        ncol = jnp.sum(g * eye_f, axis=1, keepdims=True)
        nrow = jnp.sum(g * eye_f, axis=0, keepdims=True)
        sq = ncol + nrow - 2.0 * g
        pos = sq > 0
        dist = jnp.where(pos, jnp.sqrt(jnp.where(pos, sq, 1.0)), 0.0)
        pres = (ccnt > 0).astype(jnp.float32)
        pm = _dot11(pres, pres) * noteye_f
        hinge = jnp.maximum(MARGIN_ - dist, 0.0)
        n_pairs = jnp.sum(pm)
        n_present = jnp.sum(pres)
        inter_b = jnp.where(
            n_present > 1,
            jnp.sum(hinge * pm) / jnp.maximum(n_pairs, 1.0),
            0.0)
        total_inter = total_inter + inter_b

    avg_intra = jnp.where(batch_count > 0,
                          total_intra / jnp.maximum(batch_count, 1.0), 0.0)
    avg_inter = jnp.where(batch_count > 0,
                          total_inter / jnp.maximum(batch_count, 1.0), 0.0)
    out_ref[...] = (LOSS_W_ * (avg_intra + avg_inter)).reshape(1, 1)


def _tc_finalize(fs, cnt, sqp, centers):
    return pl.pallas_call(
        _final_body,
        out_shape=jax.ShapeDtypeStruct((1, 1), jnp.float32),
    )(fs, cnt, sqp, centers)


def kernel(pred, target, feat, batch, centers):
    del pred
    tgt2 = target.reshape(N // 128, 128)
    bat2 = batch.reshape(N // 128, 128)
    fs = _sc_stage1(feat, tgt2, bat2)
    sqp, cntp = _tc_sumsq(feat, batch.reshape(N, 1), target.reshape(N, 1))
    out = _tc_finalize(fs, cntp, sqp, centers)
    return out[0, 0]


# speedup vs baseline: 11.6726x; 1.9526x over previous
import functools

import jax
import jax.numpy as jnp
from jax import lax
from jax.experimental import pallas as pl
from jax.experimental.pallas import tpu as pltpu
from jax.experimental.pallas import tpu_sc as plsc

N = 262144
D = 64
C = 16
B = 8
MARGIN_ = 0.5
LOSS_W_ = 0.01

NCORES = 2
NSUB = 16
NW = NCORES * NSUB
PW = N // NW
IDXROWS = PW // 128
CHP = 256
NSC = PW // CHP
GPC = CHP // 128


def _sc_stage1(feat, tgt2, bat2):
    mesh = plsc.VectorSubcoreMesh(core_axis_name="c", subcore_axis_name="s")
    out_type = jax.ShapeDtypeStruct((NCORES, B * C, D), jnp.float32)
    scratch = [
        pltpu.VMEM((CHP, D), jnp.float32),
        pltpu.VMEM((CHP, D), jnp.float32),
        pltpu.VMEM((IDXROWS, 128), jnp.int32),
        pltpu.VMEM((IDXROWS, 128), jnp.int32),
        pltpu.VMEM((IDXROWS, 128), jnp.int32),
        pltpu.VMEM((8, D), jnp.float32),
        pltpu.VMEM_SHARED((B * C, D), jnp.float32),
        pltpu.SemaphoreType.DMA,
        pltpu.SemaphoreType.DMA,
        pltpu.SemaphoreType.DMA,
        pltpu.SemaphoreType.DMA,
    ]

    @functools.partial(pl.kernel, out_type=out_type, mesh=mesh,
                       scratch_types=scratch)
    def k(feat_h, tgt_h, bat_h, fs_o, fA, fB, tgtv, batv, segv, zv, fs_sh,
          lsemA, lsemB, ssemA, ssemB):
        cid = lax.axis_index("c")
        sid = lax.axis_index("s")
        zero16 = jnp.zeros((16,), jnp.float32)

        for r in range(8):
            for j in range(D // 16):
                zv[r, pl.ds(16 * j, 16)] = zero16

        pltpu.sync_copy(zv, fs_sh.at[pl.ds(sid * 8, 8)])

        wid = cid * NSUB + sid
        base = wid * PW

        row0 = pl.multiple_of(wid * IDXROWS, 8)
        pltpu.sync_copy(tgt_h.at[pl.ds(row0, IDXROWS)], tgtv)
        pltpu.sync_copy(bat_h.at[pl.ds(row0, IDXROWS)], batv)

        def segrow(r, carry):
            for j in range(8):
                sl = pl.ds(16 * j, 16)
                segv[r, sl] = batv[r, sl] * C + tgtv[r, sl]
            return carry
        lax.fori_loop(0, IDXROWS, segrow, 0)
        plsc.subcore_barrier()

        def feat_src(c):
            pt0 = pl.multiple_of(base + c * CHP, CHP)
            return feat_h.at[pl.ds(pt0, CHP)]

        def ld_start(c, buf, sem):
            pltpu.async_copy(feat_src(c), buf, sem)

        def ld_wait(c, buf, sem):
            pltpu.make_async_copy(feat_src(c), buf, sem).wait()

        def scat_start(c, buf, sem):
            for g in range(GPC):
                pltpu.async_copy(buf.at[pl.ds(g * 128, 128)],
                                 fs_sh.at[segv.at[c * GPC + g]], sem, add=True)

        def scat_wait(c, buf, sem):
            for g in range(GPC):
                pltpu.make_async_copy(buf.at[pl.ds(g * 128, 128)],
                                      fs_sh.at[segv.at[c * GPC + g]],
                                      sem).wait()

        ld_start(0, fA, lsemA)

        def pipe(i, carry):
            for b in range(2):
                c = i * 2 + b
                buf, lsem, ssem = (fA, lsemA, ssemA) if b == 0 else (fB, lsemB, ssemB)
                obuf, olsem, ossem = (fB, lsemB, ssemB) if b == 0 else (fA, lsemA, ssemA)
                ld_wait(c, buf, lsem)

                @pl.when(c >= 1)
                def _():
                    scat_wait(c - 1, obuf, ossem)

                @pl.when(c + 1 < NSC)
                def _():
                    ld_start(c + 1, obuf, olsem)

                scat_start(c, buf, ssem)
            return carry
        lax.fori_loop(0, NSC // 2, pipe, 0)
        scat_wait(NSC - 1, fB, ssemB)

        plsc.subcore_barrier()

        @pl.when(sid == 0)
        def _():
            pltpu.sync_copy(fs_sh, fs_o.at[cid])

    return k(feat, tgt2, bat2)


_BLK = 8192
_NBLK = N // _BLK


def _sumsq_body(feat_ref, bat_ref, tgt_ref, sq_ref, cnt_ref):
    f = feat_ref[...]
    g2 = f * f
    rs = lax.dot_general(g2, jnp.ones((D, 1), jnp.float32),
                         (((1,), (0,)), ((), ())))
    bids = bat_ref[...].reshape(1, _BLK)
    tids = tgt_ref[...].reshape(1, _BLK)
    ohb = (bids == lax.broadcasted_iota(jnp.int32, (B, 1), 0)).astype(jnp.float32)
    ohc = (tids == lax.broadcasted_iota(jnp.int32, (C, 1), 0)).astype(jnp.float32)
    sq8 = lax.dot_general(ohb, rs, (((1,), (0,)), ((), ())))
    cntbc = lax.dot_general(ohb, ohc, (((1,), (1,)), ((), ())))
    sq_ref[...] = sq8.reshape(1, B, 1)
    cnt_ref[...] = cntbc.reshape(1, B, C)


def _tc_sumsq(feat, batch, target):
    return pl.pallas_call(
        _sumsq_body,
        grid=(_NBLK,),
        in_specs=[
            pl.BlockSpec((_BLK, D), lambda i: (i, 0)),
            pl.BlockSpec((_BLK,), lambda i: (i,)),
            pl.BlockSpec((_BLK,), lambda i: (i,)),
        ],
        out_specs=[
            pl.BlockSpec((1, B, 1), lambda i: (i, 0, 0)),
            pl.BlockSpec((1, B, C), lambda i: (i, 0, 0)),
        ],
        out_shape=[
            jax.ShapeDtypeStruct((_NBLK, B, 1), jnp.float32),
            jax.ShapeDtypeStruct((_NBLK, B, C), jnp.float32),
        ],
    )(feat, batch, target)


def _dot00(a, b):
    return lax.dot_general(a, b, (((0,), (0,)), ((), ())))


def _dot11(a, b):
    return lax.dot_general(a, b, (((1,), (1,)), ((), ())))


def _final_body(fs_ref, cnt_ref, sqp_ref, cen_ref, out_ref):
    fs2 = fs_ref[...]
    fs = fs2[0] + fs2[1]
    cnt_bc = jnp.sum(cnt_ref[...], axis=0)
    sqb8 = jnp.sum(sqp_ref[...], axis=0)
    cen = cen_ref[...]
    cen_t = jnp.concatenate([cen] * B, axis=0)

    ohg = (lax.broadcasted_iota(jnp.int32, (B * C, B), 0) // C
           == lax.broadcasted_iota(jnp.int32, (B * C, B), 1)).astype(jnp.float32)
    m128 = (lax.broadcasted_iota(jnp.int32, (B * C, C), 0) % C
            == lax.broadcasted_iota(jnp.int32, (B * C, C), 1)).astype(jnp.float32)
    cnt = jnp.sum(jnp.dot(ohg, cnt_bc) * m128, axis=1, keepdims=True)

    rowdot = jnp.sum(fs * cen_t, axis=1, keepdims=True)
    rowcn2 = jnp.sum(cen_t * cen_t, axis=1, keepdims=True)
    dotterm = _dot00(ohg, rowdot)
    cterm = _dot00(ohg, cnt * rowcn2)
    cnt_b = _dot00(ohg, cnt)
    intra_sum = sqb8 - 2.0 * dotterm + cterm
    has_b = cnt_b > 0
    intra_b = jnp.where(has_b, intra_sum / jnp.maximum(cnt_b, 1.0), 0.0)
    total_intra = jnp.sum(intra_b)
    batch_count = jnp.sum(has_b.astype(jnp.float32))

    ri = lax.broadcasted_iota(jnp.int32, (C, C), 0)
    ci = lax.broadcasted_iota(jnp.int32, (C, C), 1)
    eye_f = (ri == ci).astype(jnp.float32)
    noteye_f = 1.0 - eye_f
    total_inter = jnp.float32(0.0)
    for b in range(B):
        ccnt = cnt[b * C:(b + 1) * C, :]
        cc = fs[b * C:(b + 1) * C, :] / jnp.maximum(ccnt, 1.0)
        g = _dot11(cc, cc)
        ncol = jnp.sum(g * eye_f, axis=1, keepdims=True)
        nrow = jnp.sum(g * eye_f, axis=0, keepdims=True)
        sq = ncol + nrow - 2.0 * g
        pos = sq > 0
        dist = jnp.where(pos, jnp.sqrt(jnp.where(pos, sq, 1.0)), 0.0)
        pres = (ccnt > 0).astype(jnp.float32)
        pm = _dot11(pres, pres) * noteye_f
        hinge = jnp.maximum(MARGIN_ - dist, 0.0)
        n_pairs = jnp.sum(pm)
        n_present = jnp.sum(pres)
        inter_b = jnp.where(
            n_present > 1,
            jnp.sum(hinge * pm) / jnp.maximum(n_pairs, 1.0),
            0.0)
        total_inter = total_inter + inter_b

    avg_intra = jnp.where(batch_count > 0,
                          total_intra / jnp.maximum(batch_count, 1.0), 0.0)
    avg_inter = jnp.where(batch_count > 0,
                          total_inter / jnp.maximum(batch_count, 1.0), 0.0)
    out_ref[...] = (LOSS_W_ * (avg_intra + avg_inter)).reshape(1, 1)


def _tc_finalize(fs, cnt, sqp, centers):
    return pl.pallas_call(
        _final_body,
        out_shape=jax.ShapeDtypeStruct((1, 1), jnp.float32),
    )(fs, cnt, sqp, centers)


def kernel(pred, target, feat, batch, centers):
    del pred
    tgt2 = target.reshape(N // 128, 128)
    bat2 = batch.reshape(N // 128, 128)
    fs = _sc_stage1(feat, tgt2, bat2)
    sqp, cntp = _tc_sumsq(feat, batch, target)
    out = _tc_finalize(fs, cntp, sqp, centers)
    return out[0, 0]


# speedup vs baseline: 15.0414x; 1.2886x over previous
import functools

import jax
import jax.numpy as jnp
from jax import lax
from jax.experimental import pallas as pl
from jax.experimental.pallas import tpu as pltpu
from jax.experimental.pallas import tpu_sc as plsc

N = 262144
D = 64
C = 16
B = 8
MARGIN_ = 0.5
LOSS_W_ = 0.01

NCORES = 2
NSUB = 16
NW = NCORES * NSUB
SC_N = N // 2
PW = SC_N // NW
IDXROWS = PW // 128
CHP = 256
NSC = PW // CHP
GPC = CHP // 128


def _sc_stage1(feat, tgt2, bat2):
    mesh = plsc.VectorSubcoreMesh(core_axis_name="c", subcore_axis_name="s")
    out_type = jax.ShapeDtypeStruct((NCORES, B * C, D), jnp.float32)
    scratch = [
        pltpu.VMEM((CHP, D), jnp.float32),
        pltpu.VMEM((CHP, D), jnp.float32),
        pltpu.VMEM((IDXROWS, 128), jnp.int32),
        pltpu.VMEM((IDXROWS, 128), jnp.int32),
        pltpu.VMEM((IDXROWS, 128), jnp.int32),
        pltpu.VMEM((8, D), jnp.float32),
        pltpu.VMEM_SHARED((B * C, D), jnp.float32),
        pltpu.SemaphoreType.DMA,
        pltpu.SemaphoreType.DMA,
        pltpu.SemaphoreType.DMA,
        pltpu.SemaphoreType.DMA,
    ]

    @functools.partial(pl.kernel, out_type=out_type, mesh=mesh,
                       scratch_types=scratch)
    def k(feat_h, tgt_h, bat_h, fs_o, fA, fB, tgtv, batv, segv, zv, fs_sh,
          lsemA, lsemB, ssemA, ssemB):
        cid = lax.axis_index("c")
        sid = lax.axis_index("s")
        zero16 = jnp.zeros((16,), jnp.float32)

        for r in range(8):
            for j in range(D // 16):
                zv[r, pl.ds(16 * j, 16)] = zero16

        pltpu.sync_copy(zv, fs_sh.at[pl.ds(sid * 8, 8)])

        wid = cid * NSUB + sid
        base = wid * PW

        row0 = pl.multiple_of(wid * IDXROWS, 8)
        pltpu.sync_copy(tgt_h.at[pl.ds(row0, IDXROWS)], tgtv)
        pltpu.sync_copy(bat_h.at[pl.ds(row0, IDXROWS)], batv)

        def segrow(r, carry):
            for j in range(8):
                sl = pl.ds(16 * j, 16)
                segv[r, sl] = batv[r, sl] * C + tgtv[r, sl]
            return carry
        lax.fori_loop(0, IDXROWS, segrow, 0)
        plsc.subcore_barrier()

        def feat_src(c):
            pt0 = pl.multiple_of(base + c * CHP, CHP)
            return feat_h.at[pl.ds(pt0, CHP)]

        def ld_start(c, buf, sem):
            pltpu.async_copy(feat_src(c), buf, sem)

        def ld_wait(c, buf, sem):
            pltpu.make_async_copy(feat_src(c), buf, sem).wait()

        def scat_start(c, buf, sem):
            for g in range(GPC):
                pltpu.async_copy(buf.at[pl.ds(g * 128, 128)],
                                 fs_sh.at[segv.at[c * GPC + g]], sem, add=True)

        def scat_wait(c, buf, sem):
            for g in range(GPC):
                pltpu.make_async_copy(buf.at[pl.ds(g * 128, 128)],
                                      fs_sh.at[segv.at[c * GPC + g]],
                                      sem).wait()

        ld_start(0, fA, lsemA)

        def pipe(i, carry):
            for b in range(2):
                c = i * 2 + b
                buf, lsem, ssem = (fA, lsemA, ssemA) if b == 0 else (fB, lsemB, ssemB)
                obuf, olsem, ossem = (fB, lsemB, ssemB) if b == 0 else (fA, lsemA, ssemA)
                ld_wait(c, buf, lsem)

                @pl.when(c >= 1)
                def _():
                    scat_wait(c - 1, obuf, ossem)

                @pl.when(c + 1 < NSC)
                def _():
                    ld_start(c + 1, obuf, olsem)

                scat_start(c, buf, ssem)
            return carry
        lax.fori_loop(0, NSC // 2, pipe, 0)
        scat_wait(NSC - 1, fB, ssemB)

        plsc.subcore_barrier()

        @pl.when(sid == 0)
        def _():
            pltpu.sync_copy(fs_sh, fs_o.at[cid])

    return k(feat, tgt2, bat2)


_BLK = 8192
_NBLK = N // _BLK


def _sumsq_body(ft_ref, bat_ref, tgt_ref, sq_ref, cnt_ref):
    ft = ft_ref[...]
    g2 = ft * ft
    rs = lax.dot_general(jnp.ones((1, D), jnp.float32), g2,
                         (((1,), (0,)), ((), ())))
    bids = bat_ref[...].reshape(1, _BLK)
    tids = tgt_ref[...].reshape(1, _BLK)
    ohb = (bids == lax.broadcasted_iota(jnp.int32, (B, 1), 0)).astype(jnp.float32)
    ohc = (tids == lax.broadcasted_iota(jnp.int32, (C, 1), 0)).astype(jnp.float32)
    sq8 = lax.dot_general(ohb, rs, (((1,), (1,)), ((), ())))
    cntbc = lax.dot_general(ohb, ohc, (((1,), (1,)), ((), ())))
    sq_ref[...] = sq8.reshape(1, B, 1)
    cnt_ref[...] = cntbc.reshape(1, B, C)


def _tc_sumsq(ft, batch, target):
    return pl.pallas_call(
        _sumsq_body,
        grid=(_NBLK,),
        in_specs=[
            pl.BlockSpec((D, _BLK), lambda i: (0, i)),
            pl.BlockSpec((_BLK,), lambda i: (i,)),
            pl.BlockSpec((_BLK,), lambda i: (i,)),
        ],
        out_specs=[
            pl.BlockSpec((1, B, 1), lambda i: (i, 0, 0)),
            pl.BlockSpec((1, B, C), lambda i: (i, 0, 0)),
        ],
        out_shape=[
            jax.ShapeDtypeStruct((_NBLK, B, 1), jnp.float32),
            jax.ShapeDtypeStruct((_NBLK, B, C), jnp.float32),
        ],
    )(ft, batch, target)


_FSBLK = 8192
_FSNBLK = (N - SC_N) // _FSBLK


def _fs_tc_body(ft_ref, bat_ref, tgt_ref, out_ref):
    i = pl.program_id(0)
    ft = ft_ref[...]
    bids = bat_ref[...].reshape(1, _FSBLK)
    tids = tgt_ref[...].reshape(1, _FSBLK)
    seg = bids * C + tids
    ohs = (seg == lax.broadcasted_iota(jnp.int32, (B * C, 1), 0)
           ).astype(jnp.float32)
    part = lax.dot_general(ohs, ft, (((1,), (1,)), ((), ())))

    @pl.when(i == 0)
    def _():
        out_ref[...] = jnp.zeros((B * C, D), jnp.float32)
    out_ref[...] += part


def _fs_tc(ft_half, bat_half, tgt_half):
    return pl.pallas_call(
        _fs_tc_body,
        grid=(_FSNBLK,),
        in_specs=[
            pl.BlockSpec((D, _FSBLK), lambda i: (0, i)),
            pl.BlockSpec((_FSBLK,), lambda i: (i,)),
            pl.BlockSpec((_FSBLK,), lambda i: (i,)),
        ],
        out_specs=pl.BlockSpec((B * C, D), lambda i: (0, 0)),
        out_shape=jax.ShapeDtypeStruct((B * C, D), jnp.float32),
    )(ft_half, bat_half, tgt_half)


def _dot00(a, b):
    return lax.dot_general(a, b, (((0,), (0,)), ((), ())))


def _dot11(a, b):
    return lax.dot_general(a, b, (((1,), (1,)), ((), ())))


def _final_body(fs_ref, fstc_ref, cnt_ref, sqp_ref, cen_ref, out_ref):
    fs2 = fs_ref[...]
    fs = fs2[0] + fs2[1] + fstc_ref[...]
    cnt_bc = jnp.sum(cnt_ref[...], axis=0)
    sqb8 = jnp.sum(sqp_ref[...], axis=0)
    cen = cen_ref[...]
    cen_t = jnp.concatenate([cen] * B, axis=0)

    ohg = (lax.broadcasted_iota(jnp.int32, (B * C, B), 0) // C
           == lax.broadcasted_iota(jnp.int32, (B * C, B), 1)).astype(jnp.float32)
    m128 = (lax.broadcasted_iota(jnp.int32, (B * C, C), 0) % C
            == lax.broadcasted_iota(jnp.int32, (B * C, C), 1)).astype(jnp.float32)
    cnt = jnp.sum(jnp.dot(ohg, cnt_bc) * m128, axis=1, keepdims=True)

    rowdot = jnp.sum(fs * cen_t, axis=1, keepdims=True)
    rowcn2 = jnp.sum(cen_t * cen_t, axis=1, keepdims=True)
    dotterm = _dot00(ohg, rowdot)
    cterm = _dot00(ohg, cnt * rowcn2)
    cnt_b = _dot00(ohg, cnt)
    intra_sum = sqb8 - 2.0 * dotterm + cterm
    has_b = cnt_b > 0
    intra_b = jnp.where(has_b, intra_sum / jnp.maximum(cnt_b, 1.0), 0.0)
    total_intra = jnp.sum(intra_b)
    batch_count = jnp.sum(has_b.astype(jnp.float32))

    ri = lax.broadcasted_iota(jnp.int32, (C, C), 0)
    ci = lax.broadcasted_iota(jnp.int32, (C, C), 1)
    eye_f = (ri == ci).astype(jnp.float32)
    noteye_f = 1.0 - eye_f
    total_inter = jnp.float32(0.0)
    for b in range(B):
        ccnt = cnt[b * C:(b + 1) * C, :]
        cc = fs[b * C:(b + 1) * C, :] / jnp.maximum(ccnt, 1.0)
        g = _dot11(cc, cc)
        ncol = jnp.sum(g * eye_f, axis=1, keepdims=True)
        nrow = jnp.sum(g * eye_f, axis=0, keepdims=True)
        sq = ncol + nrow - 2.0 * g
        pos = sq > 0
        dist = jnp.where(pos, jnp.sqrt(jnp.where(pos, sq, 1.0)), 0.0)
        pres = (ccnt > 0).astype(jnp.float32)
        pm = _dot11(pres, pres) * noteye_f
        hinge = jnp.maximum(MARGIN_ - dist, 0.0)
        n_pairs = jnp.sum(pm)
        n_present = jnp.sum(pres)
        inter_b = jnp.where(
            n_present > 1,
            jnp.sum(hinge * pm) / jnp.maximum(n_pairs, 1.0),
            0.0)
        total_inter = total_inter + inter_b

    avg_intra = jnp.where(batch_count > 0,
                          total_intra / jnp.maximum(batch_count, 1.0), 0.0)
    avg_inter = jnp.where(batch_count > 0,
                          total_inter / jnp.maximum(batch_count, 1.0), 0.0)
    out_ref[...] = (LOSS_W_ * (avg_intra + avg_inter)).reshape(1, 1)


def _tc_finalize(fs, fs_tc, cnt, sqp, centers):
    return pl.pallas_call(
        _final_body,
        out_shape=jax.ShapeDtypeStruct((1, 1), jnp.float32),
    )(fs, fs_tc, cnt, sqp, centers)


def kernel(pred, target, feat, batch, centers):
    del pred
    ft = feat.T
    tgt2 = target[N - SC_N:].reshape(SC_N // 128, 128)
    bat2 = batch[N - SC_N:].reshape(SC_N // 128, 128)
    fs = _sc_stage1(feat[N - SC_N:], tgt2, bat2)
    sqp, cntp = _tc_sumsq(ft, batch, target)
    fs_tc = _fs_tc(ft, batch, target)
    out = _tc_finalize(fs, fs_tc, cntp, sqp, centers)
    return out[0, 0]


# speedup vs baseline: 21.0869x; 1.4019x over previous
import functools

import jax
import jax.numpy as jnp
from jax import lax
from jax.experimental import pallas as pl
from jax.experimental.pallas import tpu as pltpu
from jax.experimental.pallas import tpu_sc as plsc

N = 262144
D = 64
C = 16
B = 8
MARGIN_ = 0.5
LOSS_W_ = 0.01

NCORES = 2
NSUB = 16
NW = NCORES * NSUB
SC_N = N // 2
PW = SC_N // NW
IDXROWS = PW // 128
CHP = 256
NSC = PW // CHP
GPC = CHP // 128


def _sc_stage1(feat, tgt2, bat2):
    mesh = plsc.VectorSubcoreMesh(core_axis_name="c", subcore_axis_name="s")
    out_type = jax.ShapeDtypeStruct((NCORES, B * C, D), jnp.float32)
    scratch = [
        pltpu.VMEM((CHP, D), jnp.float32),
        pltpu.VMEM((CHP, D), jnp.float32),
        pltpu.VMEM((IDXROWS, 128), jnp.int32),
        pltpu.VMEM((IDXROWS, 128), jnp.int32),
        pltpu.VMEM((IDXROWS, 128), jnp.int32),
        pltpu.VMEM((8, D), jnp.float32),
        pltpu.VMEM_SHARED((B * C, D), jnp.float32),
        pltpu.SemaphoreType.DMA,
        pltpu.SemaphoreType.DMA,
        pltpu.SemaphoreType.DMA,
        pltpu.SemaphoreType.DMA,
    ]

    @functools.partial(pl.kernel, out_type=out_type, mesh=mesh,
                       scratch_types=scratch)
    def k(feat_h, tgt_h, bat_h, fs_o, fA, fB, tgtv, batv, segv, zv, fs_sh,
          lsemA, lsemB, ssemA, ssemB):
        cid = lax.axis_index("c")
        sid = lax.axis_index("s")
        zero16 = jnp.zeros((16,), jnp.float32)

        for r in range(8):
            for j in range(D // 16):
                zv[r, pl.ds(16 * j, 16)] = zero16

        pltpu.sync_copy(zv, fs_sh.at[pl.ds(sid * 8, 8)])

        wid = cid * NSUB + sid
        base = wid * PW

        row0 = pl.multiple_of(wid * IDXROWS, 8)
        pltpu.sync_copy(tgt_h.at[pl.ds(row0, IDXROWS)], tgtv)
        pltpu.sync_copy(bat_h.at[pl.ds(row0, IDXROWS)], batv)

        def segrow(r, carry):
            for j in range(8):
                sl = pl.ds(16 * j, 16)
                segv[r, sl] = batv[r, sl] * C + tgtv[r, sl]
            return carry
        lax.fori_loop(0, IDXROWS, segrow, 0)
        plsc.subcore_barrier()

        def feat_src(c):
            pt0 = pl.multiple_of(base + c * CHP, CHP)
            return feat_h.at[pl.ds(pt0, CHP)]

        def ld_start(c, buf, sem):
            pltpu.async_copy(feat_src(c), buf, sem)

        def ld_wait(c, buf, sem):
            pltpu.make_async_copy(feat_src(c), buf, sem).wait()

        def scat_start(c, buf, sem):
            for g in range(GPC):
                pltpu.async_copy(buf.at[pl.ds(g * 128, 128)],
                                 fs_sh.at[segv.at[c * GPC + g]], sem, add=True)

        def scat_wait(c, buf, sem):
            for g in range(GPC):
                pltpu.make_async_copy(buf.at[pl.ds(g * 128, 128)],
                                      fs_sh.at[segv.at[c * GPC + g]],
                                      sem).wait()

        ld_start(0, fA, lsemA)

        def pipe(i, carry):
            for b in range(2):
                c = i * 2 + b
                buf, lsem, ssem = (fA, lsemA, ssemA) if b == 0 else (fB, lsemB, ssemB)
                obuf, olsem, ossem = (fB, lsemB, ssemB) if b == 0 else (fA, lsemA, ssemA)
                ld_wait(c, buf, lsem)

                @pl.when(c >= 1)
                def _():
                    scat_wait(c - 1, obuf, ossem)

                @pl.when(c + 1 < NSC)
                def _():
                    ld_start(c + 1, obuf, olsem)

                scat_start(c, buf, ssem)
            return carry
        lax.fori_loop(0, NSC // 2, pipe, 0)
        scat_wait(NSC - 1, fB, ssemB)

        plsc.subcore_barrier()

        @pl.when(sid == 0)
        def _():
            pltpu.sync_copy(fs_sh, fs_o.at[cid])

    return k(feat, tgt2, bat2)


_BLK = 8192
_NBLK = N // _BLK


def _stats(ft, bids, tids, nb):
    g2 = ft * ft
    rs = lax.dot_general(jnp.ones((1, D), jnp.float32), g2,
                         (((1,), (0,)), ((), ())))
    brow = bids.reshape(1, nb)
    trow = tids.reshape(1, nb)
    ohb = (brow == lax.broadcasted_iota(jnp.int32, (B, 1), 0)).astype(jnp.float32)
    ohc = (trow == lax.broadcasted_iota(jnp.int32, (C, 1), 0)).astype(jnp.float32)
    sq8 = lax.dot_general(ohb, rs, (((1,), (1,)), ((), ())))
    cntbc = lax.dot_general(ohb, ohc, (((1,), (1,)), ((), ())))
    return sq8, cntbc, ohb, ohc


_TBLK = 4096
_TNBLK = SC_N // _TBLK
_TOFF = (N - SC_N) // _TBLK


def _t2_body(ft_ref, bat_ref, tgt_ref, out_ref, sq_ref, cnt_ref):
    ft = ft_ref[...]
    out_ref[...] = ft.T
    sq8, cntbc, _, _ = _stats(ft, bat_ref[...], tgt_ref[...], _TBLK)
    sq_ref[...] = sq8.reshape(1, B, 1)
    cnt_ref[...] = cntbc.reshape(1, B, C)


def _tc_transpose_stats(ft, batch, target):
    return pl.pallas_call(
        _t2_body,
        grid=(_TNBLK,),
        in_specs=[
            pl.BlockSpec((D, _TBLK), lambda i: (0, i + _TOFF)),
            pl.BlockSpec((_TBLK,), lambda i: (i + _TOFF,)),
            pl.BlockSpec((_TBLK,), lambda i: (i + _TOFF,)),
        ],
        out_specs=[
            pl.BlockSpec((_TBLK, D), lambda i: (i, 0)),
            pl.BlockSpec((1, B, 1), lambda i: (i, 0, 0)),
            pl.BlockSpec((1, B, C), lambda i: (i, 0, 0)),
        ],
        out_shape=[
            jax.ShapeDtypeStruct((SC_N, D), jnp.float32),
            jax.ShapeDtypeStruct((_TNBLK, B, 1), jnp.float32),
            jax.ShapeDtypeStruct((_TNBLK, B, C), jnp.float32),
        ],
    )(ft, batch, target)


_FSBLK = 8192
_FSNBLK = (N - SC_N) // _FSBLK


def _fs_tc_body(ft_ref, bat_ref, tgt_ref, out_ref, sq_ref, cnt_ref):
    i = pl.program_id(0)
    ft = ft_ref[...]
    bids = bat_ref[...]
    tids = tgt_ref[...]
    sq8, cntbc, ohb, ohc = _stats(ft, bids, tids, _FSBLK)
    sq_ref[...] = sq8.reshape(1, B, 1)
    cnt_ref[...] = cntbc.reshape(1, B, C)
    seg = bids.reshape(1, _FSBLK) * C + tids.reshape(1, _FSBLK)
    ohs = (seg == lax.broadcasted_iota(jnp.int32, (B * C, 1), 0)
           ).astype(jnp.float32)
    part = lax.dot_general(ohs, ft, (((1,), (1,)), ((), ())))

    @pl.when(i == 0)
    def _():
        out_ref[...] = jnp.zeros((B * C, D), jnp.float32)
    out_ref[...] += part


def _fs_tc(ft, batch, target):
    return pl.pallas_call(
        _fs_tc_body,
        grid=(_FSNBLK,),
        in_specs=[
            pl.BlockSpec((D, _FSBLK), lambda i: (0, i)),
            pl.BlockSpec((_FSBLK,), lambda i: (i,)),
            pl.BlockSpec((_FSBLK,), lambda i: (i,)),
        ],
        out_specs=[
            pl.BlockSpec((B * C, D), lambda i: (0, 0)),
            pl.BlockSpec((1, B, 1), lambda i: (i, 0, 0)),
            pl.BlockSpec((1, B, C), lambda i: (i, 0, 0)),
        ],
        out_shape=[
            jax.ShapeDtypeStruct((B * C, D), jnp.float32),
            jax.ShapeDtypeStruct((_FSNBLK, B, 1), jnp.float32),
            jax.ShapeDtypeStruct((_FSNBLK, B, C), jnp.float32),
        ],
    )(ft, batch, target)


def _dot00(a, b):
    return lax.dot_general(a, b, (((0,), (0,)), ((), ())))


def _dot11(a, b):
    return lax.dot_general(a, b, (((1,), (1,)), ((), ())))


def _final_body(fs_ref, fstc_ref, cnt_ref, sqp_ref, cen_ref, out_ref):
    fs2 = fs_ref[...]
    fs = fs2[0] + fs2[1] + fstc_ref[...]
    cnt_bc = jnp.sum(cnt_ref[...], axis=0)
    sqb8 = jnp.sum(sqp_ref[...], axis=0)
    cen = cen_ref[...]
    cen_t = jnp.concatenate([cen] * B, axis=0)

    ohg = (lax.broadcasted_iota(jnp.int32, (B * C, B), 0) // C
           == lax.broadcasted_iota(jnp.int32, (B * C, B), 1)).astype(jnp.float32)
    m128 = (lax.broadcasted_iota(jnp.int32, (B * C, C), 0) % C
            == lax.broadcasted_iota(jnp.int32, (B * C, C), 1)).astype(jnp.float32)
    cnt = jnp.sum(jnp.dot(ohg, cnt_bc) * m128, axis=1, keepdims=True)

    rowdot = jnp.sum(fs * cen_t, axis=1, keepdims=True)
    rowcn2 = jnp.sum(cen_t * cen_t, axis=1, keepdims=True)
    dotterm = _dot00(ohg, rowdot)
    cterm = _dot00(ohg, cnt * rowcn2)
    cnt_b = _dot00(ohg, cnt)
    intra_sum = sqb8 - 2.0 * dotterm + cterm
    has_b = cnt_b > 0
    intra_b = jnp.where(has_b, intra_sum / jnp.maximum(cnt_b, 1.0), 0.0)
    total_intra = jnp.sum(intra_b)
    batch_count = jnp.sum(has_b.astype(jnp.float32))

    ri = lax.broadcasted_iota(jnp.int32, (C, C), 0)
    ci = lax.broadcasted_iota(jnp.int32, (C, C), 1)
    eye_f = (ri == ci).astype(jnp.float32)
    noteye_f = 1.0 - eye_f
    total_inter = jnp.float32(0.0)
    for b in range(B):
        ccnt = cnt[b * C:(b + 1) * C, :]
        cc = fs[b * C:(b + 1) * C, :] / jnp.maximum(ccnt, 1.0)
        g = _dot11(cc, cc)
        ncol = jnp.sum(g * eye_f, axis=1, keepdims=True)
        nrow = jnp.sum(g * eye_f, axis=0, keepdims=True)
        sq = ncol + nrow - 2.0 * g
        pos = sq > 0
        dist = jnp.where(pos, jnp.sqrt(jnp.where(pos, sq, 1.0)), 0.0)
        pres = (ccnt > 0).astype(jnp.float32)
        pm = _dot11(pres, pres) * noteye_f
        hinge = jnp.maximum(MARGIN_ - dist, 0.0)
        n_pairs = jnp.sum(pm)
        n_present = jnp.sum(pres)
        inter_b = jnp.where(
            n_present > 1,
            jnp.sum(hinge * pm) / jnp.maximum(n_pairs, 1.0),
            0.0)
        total_inter = total_inter + inter_b

    avg_intra = jnp.where(batch_count > 0,
                          total_intra / jnp.maximum(batch_count, 1.0), 0.0)
    avg_inter = jnp.where(batch_count > 0,
                          total_inter / jnp.maximum(batch_count, 1.0), 0.0)
    out_ref[...] = (LOSS_W_ * (avg_intra + avg_inter)).reshape(1, 1)


def _tc_finalize(fs, fs_tc, cnt, sqp, centers):
    return pl.pallas_call(
        _final_body,
        out_shape=jax.ShapeDtypeStruct((1, 1), jnp.float32),
    )(fs, fs_tc, cnt, sqp, centers)


def kernel(pred, target, feat, batch, centers):
    del pred
    ft = feat.T
    tgt2 = target[N - SC_N:].reshape(SC_N // 128, 128)
    bat2 = batch[N - SC_N:].reshape(SC_N // 128, 128)
    feat_sc, sqp2, cntp2 = _tc_transpose_stats(ft, batch, target)
    fs = _sc_stage1(feat_sc, tgt2, bat2)
    fs_tc, sqp1, cntp1 = _fs_tc(ft, batch, target)
    sqp = jnp.concatenate([sqp1, sqp2], axis=0)
    cntp = jnp.concatenate([cntp1, cntp2], axis=0)
    out = _tc_finalize(fs, fs_tc, cntp, sqp, centers)
    return out[0, 0]


# speedup vs baseline: 22.8554x; 1.0839x over previous
import functools

import jax
import jax.numpy as jnp
from jax import lax
from jax.experimental import pallas as pl
from jax.experimental.pallas import tpu as pltpu
from jax.experimental.pallas import tpu_sc as plsc

N = 262144
D = 64
C = 16
B = 8
MARGIN_ = 0.5
LOSS_W_ = 0.01

NCORES = 2
NSUB = 16
NW = NCORES * NSUB
SC_N = N // 2
PW = SC_N // NW
IDXROWS = PW // 128
CHP = 256
NSC = PW // CHP
GPC = CHP // 128


def _sc_stage1(feat, tgt2, bat2):
    mesh = plsc.VectorSubcoreMesh(core_axis_name="c", subcore_axis_name="s")
    out_type = jax.ShapeDtypeStruct((NCORES, B * C, D), jnp.float32)
    scratch = [
        pltpu.VMEM((CHP, D), jnp.float32),
        pltpu.VMEM((CHP, D), jnp.float32),
        pltpu.VMEM((IDXROWS, 128), jnp.int32),
        pltpu.VMEM((IDXROWS, 128), jnp.int32),
        pltpu.VMEM((IDXROWS, 128), jnp.int32),
        pltpu.VMEM((8, D), jnp.float32),
        pltpu.VMEM_SHARED((B * C, D), jnp.float32),
        pltpu.SemaphoreType.DMA,
        pltpu.SemaphoreType.DMA,
        pltpu.SemaphoreType.DMA,
        pltpu.SemaphoreType.DMA,
    ]

    @functools.partial(pl.kernel, out_type=out_type, mesh=mesh,
                       scratch_types=scratch)
    def k(feat_h, tgt_h, bat_h, fs_o, fA, fB, tgtv, batv, segv, zv, fs_sh,
          lsemA, lsemB, ssemA, ssemB):
        cid = lax.axis_index("c")
        sid = lax.axis_index("s")
        zero16 = jnp.zeros((16,), jnp.float32)

        for r in range(8):
            for j in range(D // 16):
                zv[r, pl.ds(16 * j, 16)] = zero16

        pltpu.sync_copy(zv, fs_sh.at[pl.ds(sid * 8, 8)])

        wid = cid * NSUB + sid
        base = wid * PW

        row0 = pl.multiple_of(wid * IDXROWS, 8)
        pltpu.sync_copy(tgt_h.at[pl.ds(row0, IDXROWS)], tgtv)
        pltpu.sync_copy(bat_h.at[pl.ds(row0, IDXROWS)], batv)

        def segrow(r, carry):
            for j in range(8):
                sl = pl.ds(16 * j, 16)
                segv[r, sl] = batv[r, sl] * C + tgtv[r, sl]
            return carry
        lax.fori_loop(0, IDXROWS, segrow, 0)
        plsc.subcore_barrier()

        def feat_src(c):
            pt0 = pl.multiple_of(base + c * CHP, CHP)
            return feat_h.at[pl.ds(pt0, CHP)]

        def ld_start(c, buf, sem):
            pltpu.async_copy(feat_src(c), buf, sem)

        def ld_wait(c, buf, sem):
            pltpu.make_async_copy(feat_src(c), buf, sem).wait()

        def scat_start(c, buf, sem):
            for g in range(GPC):
                pltpu.async_copy(buf.at[pl.ds(g * 128, 128)],
                                 fs_sh.at[segv.at[c * GPC + g]], sem, add=True)

        def scat_wait(c, buf, sem):
            for g in range(GPC):
                pltpu.make_async_copy(buf.at[pl.ds(g * 128, 128)],
                                      fs_sh.at[segv.at[c * GPC + g]],
                                      sem).wait()

        ld_start(0, fA, lsemA)

        def pipe(i, carry):
            for b in range(2):
                c = i * 2 + b
                buf, lsem, ssem = (fA, lsemA, ssemA) if b == 0 else (fB, lsemB, ssemB)
                obuf, olsem, ossem = (fB, lsemB, ssemB) if b == 0 else (fA, lsemA, ssemA)
                ld_wait(c, buf, lsem)

                @pl.when(c >= 1)
                def _():
                    scat_wait(c - 1, obuf, ossem)

                @pl.when(c + 1 < NSC)
                def _():
                    ld_start(c + 1, obuf, olsem)

                scat_start(c, buf, ssem)
            return carry
        lax.fori_loop(0, NSC // 2, pipe, 0)
        scat_wait(NSC - 1, fB, ssemB)

        plsc.subcore_barrier()

        @pl.when(sid == 0)
        def _():
            pltpu.sync_copy(fs_sh, fs_o.at[cid])

    return k(feat, tgt2, bat2)


_BLK = 8192
_NBLK = N // _BLK


def _stats(ft, bids, tids, nb):
    g2 = ft * ft
    rs = lax.dot_general(jnp.ones((1, D), jnp.float32), g2,
                         (((1,), (0,)), ((), ())))
    brow = bids.reshape(1, nb)
    trow = tids.reshape(1, nb)
    ohb = (brow == lax.broadcasted_iota(jnp.int32, (B, 1), 0)).astype(jnp.float32)
    ohc = (trow == lax.broadcasted_iota(jnp.int32, (C, 1), 0)).astype(jnp.float32)
    sq8 = lax.dot_general(ohb, rs, (((1,), (1,)), ((), ())))
    cntbc = lax.dot_general(ohb, ohc, (((1,), (1,)), ((), ())))
    return sq8, cntbc, ohb, ohc


_TBLK = 8192
_TNBLK = SC_N // _TBLK
_TOFF = (N - SC_N) // _TBLK


def _t2_body(ft_ref, bat_ref, tgt_ref, out_ref, sq_ref, cnt_ref):
    ft = ft_ref[...]
    out_ref[...] = ft.T
    sq8, cntbc, _, _ = _stats(ft, bat_ref[...], tgt_ref[...], _TBLK)
    sq_ref[...] = sq8.reshape(1, B, 1)
    cnt_ref[...] = cntbc.reshape(1, B, C)


def _tc_transpose_stats(ft, batch, target):
    return pl.pallas_call(
        _t2_body,
        grid=(_TNBLK,),
        in_specs=[
            pl.BlockSpec((D, _TBLK), lambda i: (0, i + _TOFF)),
            pl.BlockSpec((_TBLK,), lambda i: (i + _TOFF,)),
            pl.BlockSpec((_TBLK,), lambda i: (i + _TOFF,)),
        ],
        out_specs=[
            pl.BlockSpec((_TBLK, D), lambda i: (i, 0)),
            pl.BlockSpec((1, B, 1), lambda i: (i, 0, 0)),
            pl.BlockSpec((1, B, C), lambda i: (i, 0, 0)),
        ],
        out_shape=[
            jax.ShapeDtypeStruct((SC_N, D), jnp.float32),
            jax.ShapeDtypeStruct((_TNBLK, B, 1), jnp.float32),
            jax.ShapeDtypeStruct((_TNBLK, B, C), jnp.float32),
        ],
    )(ft, batch, target)


_FSBLK = 8192
_FSNBLK = (N - SC_N) // _FSBLK


def _fs_tc_body(ft_ref, bat_ref, tgt_ref, out_ref, sq_ref, cnt_ref):
    i = pl.program_id(0)
    ft = ft_ref[...]
    bids = bat_ref[...]
    tids = tgt_ref[...]
    sq8, cntbc, ohb, ohc = _stats(ft, bids, tids, _FSBLK)
    sq_ref[...] = sq8.reshape(1, B, 1)
    cnt_ref[...] = cntbc.reshape(1, B, C)
    seg = bids.reshape(1, _FSBLK) * C + tids.reshape(1, _FSBLK)
    ohs = (seg == lax.broadcasted_iota(jnp.int32, (B * C, 1), 0)
           ).astype(jnp.float32)
    part = lax.dot_general(ohs, ft, (((1,), (1,)), ((), ())))

    @pl.when(i == 0)
    def _():
        out_ref[...] = jnp.zeros((B * C, D), jnp.float32)
    out_ref[...] += part


def _fs_tc(ft, batch, target):
    return pl.pallas_call(
        _fs_tc_body,
        grid=(_FSNBLK,),
        in_specs=[
            pl.BlockSpec((D, _FSBLK), lambda i: (0, i)),
            pl.BlockSpec((_FSBLK,), lambda i: (i,)),
            pl.BlockSpec((_FSBLK,), lambda i: (i,)),
        ],
        out_specs=[
            pl.BlockSpec((B * C, D), lambda i: (0, 0)),
            pl.BlockSpec((1, B, 1), lambda i: (i, 0, 0)),
            pl.BlockSpec((1, B, C), lambda i: (i, 0, 0)),
        ],
        out_shape=[
            jax.ShapeDtypeStruct((B * C, D), jnp.float32),
            jax.ShapeDtypeStruct((_FSNBLK, B, 1), jnp.float32),
            jax.ShapeDtypeStruct((_FSNBLK, B, C), jnp.float32),
        ],
    )(ft, batch, target)


def _dot00(a, b):
    return lax.dot_general(a, b, (((0,), (0,)), ((), ())))


def _dot11(a, b):
    return lax.dot_general(a, b, (((1,), (1,)), ((), ())))


def _final_body(fs_ref, fstc_ref, cnt_ref, sqp_ref, cen_ref, out_ref):
    fs2 = fs_ref[...]
    fs = fs2[0] + fs2[1] + fstc_ref[...]
    cnt_bc = jnp.sum(cnt_ref[...], axis=0)
    sqb8 = jnp.sum(sqp_ref[...], axis=0)
    cen = cen_ref[...]
    cen_t = jnp.concatenate([cen] * B, axis=0)

    ohg = (lax.broadcasted_iota(jnp.int32, (B * C, B), 0) // C
           == lax.broadcasted_iota(jnp.int32, (B * C, B), 1)).astype(jnp.float32)
    m128 = (lax.broadcasted_iota(jnp.int32, (B * C, C), 0) % C
            == lax.broadcasted_iota(jnp.int32, (B * C, C), 1)).astype(jnp.float32)
    cnt = jnp.sum(jnp.dot(ohg, cnt_bc) * m128, axis=1, keepdims=True)

    rowdot = jnp.sum(fs * cen_t, axis=1, keepdims=True)
    rowcn2 = jnp.sum(cen_t * cen_t, axis=1, keepdims=True)
    dotterm = _dot00(ohg, rowdot)
    cterm = _dot00(ohg, cnt * rowcn2)
    cnt_b = _dot00(ohg, cnt)
    intra_sum = sqb8 - 2.0 * dotterm + cterm
    has_b = cnt_b > 0
    intra_b = jnp.where(has_b, intra_sum / jnp.maximum(cnt_b, 1.0), 0.0)
    total_intra = jnp.sum(intra_b)
    batch_count = jnp.sum(has_b.astype(jnp.float32))

    ri = lax.broadcasted_iota(jnp.int32, (C, C), 0)
    ci = lax.broadcasted_iota(jnp.int32, (C, C), 1)
    eye_f = (ri == ci).astype(jnp.float32)
    noteye_f = 1.0 - eye_f
    total_inter = jnp.float32(0.0)
    for b in range(B):
        ccnt = cnt[b * C:(b + 1) * C, :]
        cc = fs[b * C:(b + 1) * C, :] / jnp.maximum(ccnt, 1.0)
        g = _dot11(cc, cc)
        ncol = jnp.sum(g * eye_f, axis=1, keepdims=True)
        nrow = jnp.sum(g * eye_f, axis=0, keepdims=True)
        sq = ncol + nrow - 2.0 * g
        pos = sq > 0
        dist = jnp.where(pos, jnp.sqrt(jnp.where(pos, sq, 1.0)), 0.0)
        pres = (ccnt > 0).astype(jnp.float32)
        pm = _dot11(pres, pres) * noteye_f
        hinge = jnp.maximum(MARGIN_ - dist, 0.0)
        n_pairs = jnp.sum(pm)
        n_present = jnp.sum(pres)
        inter_b = jnp.where(
            n_present > 1,
            jnp.sum(hinge * pm) / jnp.maximum(n_pairs, 1.0),
            0.0)
        total_inter = total_inter + inter_b

    avg_intra = jnp.where(batch_count > 0,
                          total_intra / jnp.maximum(batch_count, 1.0), 0.0)
    avg_inter = jnp.where(batch_count > 0,
                          total_inter / jnp.maximum(batch_count, 1.0), 0.0)
    out_ref[...] = (LOSS_W_ * (avg_intra + avg_inter)).reshape(1, 1)


def _tc_finalize(fs, fs_tc, cnt, sqp, centers):
    return pl.pallas_call(
        _final_body,
        out_shape=jax.ShapeDtypeStruct((1, 1), jnp.float32),
    )(fs, fs_tc, cnt, sqp, centers)


def kernel(pred, target, feat, batch, centers):
    del pred
    ft = feat.T
    tgt2 = target[N - SC_N:].reshape(SC_N // 128, 128)
    bat2 = batch[N - SC_N:].reshape(SC_N // 128, 128)
    feat_sc, sqp2, cntp2 = _tc_transpose_stats(ft, batch, target)
    fs = _sc_stage1(feat_sc, tgt2, bat2)
    fs_tc, sqp1, cntp1 = _fs_tc(ft, batch, target)
    sqp = jnp.concatenate([sqp1, sqp2], axis=0)
    cntp = jnp.concatenate([cntp1, cntp2], axis=0)
    out = _tc_finalize(fs, fs_tc, cntp, sqp, centers)
    return out[0, 0]
